# Initial kernel scaffold; baseline (speedup 1.0000x reference)
import jax
import jax.numpy as jnp
from jax.experimental import pallas as pl


def kernel(p_buffer, ref):
    raise NotImplementedError("write your pallas kernel here")


# speedup vs baseline: 10.8789x; 10.8789x over previous
import functools

import jax
import jax.numpy as jnp
import numpy as np
from jax import lax
from jax.experimental import pallas as pl
from jax.experimental.pallas import tpu as pltpu
from jax.experimental.pallas import tpu_sc as plsc

B, S, C, H, W = 8, 8, 8, 128, 128
HW = H * W
SHW = S * HW
N = B * SHW

NC, NS = 2, 16
NW = NC * NS
PER_TILE = N // NW
CH = 512
CHUNKS = PER_TILE // CH

_GAMMA = np.float32(0.454545)

def _fry_mix(k0, k1, x0, x1):
    rots = ((13, 15, 26, 6), (17, 29, 16, 24))
    ks = (k0, k1, np.uint32(k0 ^ k1 ^ np.uint32(0x1BD11BDA)))
    x0 = x0 + ks[0]
    x1 = x1 + ks[1]
    for i in range(5):
        for r in rots[i % 2]:
            x0 = x0 + x1
            x1 = ((x1 << np.uint32(r)) | (x1 >> np.uint32(32 - r))) ^ x0
        x0 = x0 + ks[(i + 1) % 3]
        x1 = x1 + ks[(i + 2) % 3] + np.uint32(i + 1)
    return x0, x1


def _fry_permutation(seed, n):
    key = (np.uint32(0), np.uint32(seed))
    x = np.arange(n, dtype=np.int32)
    num_rounds = int(np.ceil(3 * np.log(max(1, n)) / np.log(np.iinfo(np.uint32).max)))
    with np.errstate(over='ignore'):
        for _ in range(num_rounds):
            b0, b1 = _fry_mix(key[0], key[1],
                              np.zeros(2, np.uint32), np.arange(2, dtype=np.uint32))
            key, sub = (b0[0], b1[0]), (b0[1], b1[1])
            s0, s1 = _fry_mix(sub[0], sub[1],
                              np.zeros(n, np.uint32), np.arange(n, dtype=np.uint32))
            x = x[np.argsort(s0 ^ s1, kind='stable')]
    return x


def _make_consts():
    sp = _fry_permutation(1, SHW).astype(np.int32)
    sb = _fry_permutation(2, N).astype(np.int32)
    k = np.arange(N, dtype=np.int32)
    gp = (k // SHW) * SHW + sp[k % SHW]
    rp = (sp & (HW - 1)).astype(np.int32)
    rb = ((sb >> 17) << 14) | (sb & (HW - 1))
    return gp, sb.astype(np.int32), rb.astype(np.int32), rp


_CONSTS = _make_consts()


def _tonemap_body(x_ref, o_ref):
    x = jnp.maximum(x_ref[...], 0.0)
    y = x / (1.0 + x)
    t = jnp.exp(_GAMMA * jnp.log(y))
    o_ref[...] = jnp.where(y > 0.0, t, 0.0)


def _tonemap(ref24):
    return pl.pallas_call(
        _tonemap_body,
        out_shape=jax.ShapeDtypeStruct((B * 3, HW), jnp.float32),
    )(ref24)


def _sc_body(ptab, r4, rpl, gp_c, sb_c, rb_c, rp_c, out,
             plane_r, plane_g, plane_b,
             p1_v, p2p_v, p2b_v, rb4_v,
             gp_v, sb_v, rb_v, rp_v, acc_v,
             sem0, sem1, sem2, sem3):
    cid = lax.axis_index("c")
    sid = lax.axis_index("s")
    wid = sid * NC + cid
    k0 = wid * PER_TILE
    bb = wid // (SHW // PER_TILE)
    j0 = (wid % (SHW // PER_TILE)) * PER_TILE

    pltpu.sync_copy(rpl.at[bb * 3 + 0], plane_r)
    pltpu.sync_copy(rpl.at[bb * 3 + 1], plane_g)
    pltpu.sync_copy(rpl.at[bb * 3 + 2], plane_b)

    i16 = lax.iota(jnp.int32, 16)
    c_p = [jnp.full((16,), ch, jnp.int32) for ch in range(C)]
    c_r = [jnp.full((16,), ch, jnp.int32) for ch in range(3)]

    def chunk_body(g, acc):
        base_k = k0 + g * CH
        base_j = j0 + g * CH
        pltpu.sync_copy(gp_c.at[pl.ds(base_k, CH)], gp_v)
        pltpu.sync_copy(sb_c.at[pl.ds(base_k, CH)], sb_v)
        pltpu.sync_copy(rb_c.at[pl.ds(base_k, CH)], rb_v)
        pltpu.sync_copy(rp_c.at[pl.ds(base_j, CH)], rp_v)
        d1 = pltpu.async_copy(ptab.at[gp_v], p2p_v, sem0)
        d2 = pltpu.async_copy(ptab.at[sb_v], p2b_v, sem1)
        d3 = pltpu.async_copy(r4.at[rb_v], rb4_v, sem2)
        d4 = pltpu.async_copy(ptab.at[pl.ds(base_k, CH)], p1_v, sem3)
        d1.wait()
        d2.wait()
        d3.wait()
        d4.wait()

        pixbase = base_j & (HW - 1)

        def step(t, acc):
            r0 = t * 16
            ridx = r0 + i16
            rp16 = rp_v[pl.ds(r0, 16)]
            ra_r = plane_r[pl.ds(pixbase + r0, 16)]
            ra_g = plane_g[pl.ds(pixbase + r0, 16)]
            ra_b = plane_b[pl.ds(pixbase + r0, 16)]
            rpr = plsc.load_gather(plane_r, [rp16])
            rpg = plsc.load_gather(plane_g, [rp16])
            rpb = plsc.load_gather(plane_b, [rp16])
            rbr = plsc.load_gather(rb4_v, [ridx, c_r[0]])
            rbg = plsc.load_gather(rb4_v, [ridx, c_r[1]])
            rbb = plsc.load_gather(rb4_v, [ridx, c_r[2]])
            srp = ((ra_r - rpr) * (ra_r - rpr)
                   + (ra_g - rpg) * (ra_g - rpg)
                   + (ra_b - rpb) * (ra_b - rpb))
            srb = ((ra_r - rbr) * (ra_r - rbr)
                   + (ra_g - rbg) * (ra_g - rbg)
                   + (ra_b - rbb) * (ra_b - rbb))
            spp = jnp.zeros((16,), jnp.float32)
            spb = jnp.zeros((16,), jnp.float32)
            for ch in range(C):
                a = plsc.load_gather(p1_v, [ridx, c_p[ch]])
                bp = plsc.load_gather(p2p_v, [ridx, c_p[ch]])
                bq = plsc.load_gather(p2b_v, [ridx, c_p[ch]])
                dp = a - bp
                db = a - bq
                spp = spp + dp * dp
                spb = spb + db * db
            e1 = spp - srp
            e2 = spb - srb
            return acc + (e1 * e1 + e2 * e2)

        return lax.fori_loop(0, CH // 16, step, acc)

    acc = lax.fori_loop(0, CHUNKS, chunk_body, jnp.zeros((16,), jnp.float32))
    acc_v[...] = acc
    pltpu.sync_copy(acc_v, out.at[wid])


@functools.partial(pl.kernel,
                   out_type=jax.ShapeDtypeStruct((NW, 16), jnp.float32),
                   mesh=plsc.VectorSubcoreMesh(core_axis_name="c",
                                               subcore_axis_name="s"),
                   compiler_params=pltpu.CompilerParams(
                       needs_layout_passes=False,
                       use_tc_tiling_on_sc=False),
                   scratch_types=[
                       pltpu.VMEM((HW,), jnp.float32),
                       pltpu.VMEM((HW,), jnp.float32),
                       pltpu.VMEM((HW,), jnp.float32),
                       pltpu.VMEM((CH, C), jnp.float32),
                       pltpu.VMEM((CH, C), jnp.float32),
                       pltpu.VMEM((CH, C), jnp.float32),
                       pltpu.VMEM((CH, C), jnp.float32),
                       pltpu.VMEM((CH,), jnp.int32),
                       pltpu.VMEM((CH,), jnp.int32),
                       pltpu.VMEM((CH,), jnp.int32),
                       pltpu.VMEM((CH,), jnp.int32),
                       pltpu.VMEM((16,), jnp.float32),
                       pltpu.SemaphoreType.DMA,
                       pltpu.SemaphoreType.DMA,
                       pltpu.SemaphoreType.DMA,
                       pltpu.SemaphoreType.DMA,
                   ])
def _sc_pass(ptab, r4, rpl, gp_c, sb_c, rb_c, rp_c, out, *rest):
    _sc_body(ptab, r4, rpl, gp_c, sb_c, rb_c, rp_c, out, *rest)


def kernel(p_buffer, ref):
    gp, sb, rb, rp = _CONSTS

    rpl = _tonemap(ref.reshape(B * 3, HW))
    r4 = jnp.pad(jnp.transpose(rpl.reshape(B, 3, HW), (0, 2, 1)),
                 ((0, 0), (0, 0), (0, 5))).reshape(B * HW, 8)
    ptab = jnp.transpose(p_buffer, (0, 1, 3, 4, 2)).reshape(N, C)

    out = _sc_pass(ptab, r4, rpl,
                   jnp.asarray(gp), jnp.asarray(sb),
                   jnp.asarray(rb), jnp.asarray(rp))
    return (np.float32(0.125) / np.float32(N)) * jnp.sum(out)


# speedup vs baseline: 19.1490x; 1.7602x over previous
import functools

import jax
import jax.numpy as jnp
import numpy as np
from jax import lax
from jax.experimental import pallas as pl
from jax.experimental.pallas import tpu as pltpu
from jax.experimental.pallas import tpu_sc as plsc

B, S, C, H, W = 8, 8, 8, 128, 128
HW = H * W
SHW = S * HW
N = B * SHW

NC, NS = 2, 16
NW = NC * NS
PER_TILE = N // NW
CH = 512
CHUNKS = PER_TILE // CH

_GAMMA = np.float32(0.454545)

def _fry_mix(k0, k1, x0, x1):
    rots = ((13, 15, 26, 6), (17, 29, 16, 24))
    ks = (k0, k1, np.uint32(k0 ^ k1 ^ np.uint32(0x1BD11BDA)))
    x0 = x0 + ks[0]
    x1 = x1 + ks[1]
    for i in range(5):
        for r in rots[i % 2]:
            x0 = x0 + x1
            x1 = ((x1 << np.uint32(r)) | (x1 >> np.uint32(32 - r))) ^ x0
        x0 = x0 + ks[(i + 1) % 3]
        x1 = x1 + ks[(i + 2) % 3] + np.uint32(i + 1)
    return x0, x1


def _fry_permutation(seed, n):
    key = (np.uint32(0), np.uint32(seed))
    x = np.arange(n, dtype=np.int32)
    num_rounds = int(np.ceil(3 * np.log(max(1, n)) / np.log(np.iinfo(np.uint32).max)))
    with np.errstate(over='ignore'):
        for _ in range(num_rounds):
            b0, b1 = _fry_mix(key[0], key[1],
                              np.zeros(2, np.uint32), np.arange(2, dtype=np.uint32))
            key, sub = (b0[0], b1[0]), (b0[1], b1[1])
            s0, s1 = _fry_mix(sub[0], sub[1],
                              np.zeros(n, np.uint32), np.arange(n, dtype=np.uint32))
            x = x[np.argsort(s0 ^ s1, kind='stable')]
    return x


def _make_consts():
    sp = _fry_permutation(1, SHW).astype(np.int32)
    sb = _fry_permutation(2, N).astype(np.int32)
    k = np.arange(N, dtype=np.int32)
    gp = (k // SHW) * SHW + sp[k % SHW]
    rp = (sp & (HW - 1)).astype(np.int32)
    rb = ((sb >> 17) << 14) | (sb & (HW - 1))
    return gp, sb.astype(np.int32), rb.astype(np.int32), rp


_CONSTS = _make_consts()


def _tonemap_body(x_ref, o_ref):
    x = jnp.maximum(x_ref[...], 0.0)
    y = x / (1.0 + x)
    t = jnp.exp(_GAMMA * jnp.log(y))
    o_ref[...] = jnp.where(y > 0.0, t, 0.0)


def _tonemap(ref24):
    return pl.pallas_call(
        _tonemap_body,
        out_shape=jax.ShapeDtypeStruct((B * 3, HW), jnp.float32),
    )(ref24)


PC = 2048
PR = 2048


def _sc_prep_body(pflat, rpl, ptab_o, r4_o, ch_v, out_v, r3_v, r8_v, sem):
    cid = lax.axis_index("c")
    sid = lax.axis_index("s")
    wid = sid * NC + cid

    i16 = lax.iota(jnp.int32, 16)
    i8 = i16 * 8

    for b in range(2):
        bs = wid * 2 + b
        base_in = bs * (C * HW)
        base_out = bs * HW * C

        def p_chunk(c, _, base_in=base_in, base_out=base_out):
            p0 = c * PC
            cps = [pltpu.async_copy(
                pflat.at[pl.ds(base_in + ch * HW + p0, PC)],
                ch_v.at[ch], sem) for ch in range(C)]
            for cp in cps:
                cp.wait()

            def step(t, _):
                for ch in range(C):
                    v = ch_v[ch, pl.ds(t * 16, 16)]
                    plsc.store_scatter(out_v, [i8 + (t * 128 + ch)], v)
                return 0

            lax.fori_loop(0, PC // 16, step, 0)
            pltpu.sync_copy(out_v, ptab_o.at[pl.ds(base_out + p0 * 8, PC * 8)])
            return 0

        lax.fori_loop(0, HW // PC, p_chunk, 0)

    bq = wid // 4
    pr0 = (wid % 4) * (HW // 4)

    def r_chunk(c, _):
        p0 = pr0 + c * PR
        cps = [pltpu.async_copy(rpl.at[bq * 3 + ch, pl.ds(p0, PR)],
                                r3_v.at[ch], sem) for ch in range(3)]
        for cp in cps:
            cp.wait()

        def step(t, _):
            for ch in range(3):
                v = r3_v[ch, pl.ds(t * 16, 16)]
                plsc.store_scatter(r8_v, [i8 + (t * 128 + ch)], v)
            return 0

        lax.fori_loop(0, PR // 16, step, 0)
        pltpu.sync_copy(r8_v, r4_o.at[pl.ds((bq * HW + p0) * 8, PR * 8)])
        return 0

    lax.fori_loop(0, (HW // 4) // PR, r_chunk, 0)


@functools.partial(pl.kernel,
                   out_type=(jax.ShapeDtypeStruct((N * C,), jnp.float32),
                             jax.ShapeDtypeStruct((B * HW * 8,), jnp.float32)),
                   mesh=plsc.VectorSubcoreMesh(core_axis_name="c",
                                               subcore_axis_name="s"),
                   compiler_params=pltpu.CompilerParams(
                       needs_layout_passes=False,
                       use_tc_tiling_on_sc=False),
                   scratch_types=[
                       pltpu.VMEM((C, PC), jnp.float32),
                       pltpu.VMEM((PC * 8,), jnp.float32),
                       pltpu.VMEM((3, PR), jnp.float32),
                       pltpu.VMEM((PR * 8,), jnp.float32),
                       pltpu.SemaphoreType.DMA,
                   ])
def _sc_prep(pflat, rpl, ptab_o, r4_o, *rest):
    _sc_prep_body(pflat, rpl, ptab_o, r4_o, *rest)


def _sc_body(ptab, r4, rpl, gp_c, sb_c, rb_c, rp_c, out,
             plane_r, plane_g, plane_b,
             p1_v, p2p_v, p2b_v, rb4_v,
             gp_v, sb_v, rb_v, rp_v, acc_v,
             sem0, sem1, sem2, sem3):
    cid = lax.axis_index("c")
    sid = lax.axis_index("s")
    wid = sid * NC + cid
    k0 = wid * PER_TILE
    bb = wid // (SHW // PER_TILE)
    j0 = (wid % (SHW // PER_TILE)) * PER_TILE

    pltpu.sync_copy(rpl.at[bb * 3 + 0], plane_r)
    pltpu.sync_copy(rpl.at[bb * 3 + 1], plane_g)
    pltpu.sync_copy(rpl.at[bb * 3 + 2], plane_b)

    i16 = lax.iota(jnp.int32, 16)
    c_p = [jnp.full((16,), ch, jnp.int32) for ch in range(C)]
    c_r = [jnp.full((16,), ch, jnp.int32) for ch in range(3)]

    def chunk_body(g, acc):
        base_k = k0 + g * CH
        base_j = j0 + g * CH
        pltpu.sync_copy(gp_c.at[pl.ds(base_k, CH)], gp_v)
        pltpu.sync_copy(sb_c.at[pl.ds(base_k, CH)], sb_v)
        pltpu.sync_copy(rb_c.at[pl.ds(base_k, CH)], rb_v)
        pltpu.sync_copy(rp_c.at[pl.ds(base_j, CH)], rp_v)
        d1 = pltpu.async_copy(ptab.at[gp_v], p2p_v, sem0)
        d2 = pltpu.async_copy(ptab.at[sb_v], p2b_v, sem1)
        d3 = pltpu.async_copy(r4.at[rb_v], rb4_v, sem2)
        d4 = pltpu.async_copy(ptab.at[pl.ds(base_k, CH)], p1_v, sem3)
        d1.wait()
        d2.wait()
        d3.wait()
        d4.wait()

        pixbase = base_j & (HW - 1)

        def step(t, acc):
            r0 = t * 16
            ridx = r0 + i16
            rp16 = rp_v[pl.ds(r0, 16)]
            ra_r = plane_r[pl.ds(pixbase + r0, 16)]
            ra_g = plane_g[pl.ds(pixbase + r0, 16)]
            ra_b = plane_b[pl.ds(pixbase + r0, 16)]
            rpr = plsc.load_gather(plane_r, [rp16])
            rpg = plsc.load_gather(plane_g, [rp16])
            rpb = plsc.load_gather(plane_b, [rp16])
            rbr = plsc.load_gather(rb4_v, [ridx, c_r[0]])
            rbg = plsc.load_gather(rb4_v, [ridx, c_r[1]])
            rbb = plsc.load_gather(rb4_v, [ridx, c_r[2]])
            srp = ((ra_r - rpr) * (ra_r - rpr)
                   + (ra_g - rpg) * (ra_g - rpg)
                   + (ra_b - rpb) * (ra_b - rpb))
            srb = ((ra_r - rbr) * (ra_r - rbr)
                   + (ra_g - rbg) * (ra_g - rbg)
                   + (ra_b - rbb) * (ra_b - rbb))
            spp = jnp.zeros((16,), jnp.float32)
            spb = jnp.zeros((16,), jnp.float32)
            for ch in range(C):
                a = plsc.load_gather(p1_v, [ridx, c_p[ch]])
                bp = plsc.load_gather(p2p_v, [ridx, c_p[ch]])
                bq = plsc.load_gather(p2b_v, [ridx, c_p[ch]])
                dp = a - bp
                db = a - bq
                spp = spp + dp * dp
                spb = spb + db * db
            e1 = spp - srp
            e2 = spb - srb
            return acc + (e1 * e1 + e2 * e2)

        return lax.fori_loop(0, CH // 16, step, acc)

    acc = lax.fori_loop(0, CHUNKS, chunk_body, jnp.zeros((16,), jnp.float32))
    acc_v[...] = acc
    pltpu.sync_copy(acc_v, out.at[wid])


@functools.partial(pl.kernel,
                   out_type=jax.ShapeDtypeStruct((NW, 16), jnp.float32),
                   mesh=plsc.VectorSubcoreMesh(core_axis_name="c",
                                               subcore_axis_name="s"),
                   compiler_params=pltpu.CompilerParams(
                       needs_layout_passes=False,
                       use_tc_tiling_on_sc=False),
                   scratch_types=[
                       pltpu.VMEM((HW,), jnp.float32),
                       pltpu.VMEM((HW,), jnp.float32),
                       pltpu.VMEM((HW,), jnp.float32),
                       pltpu.VMEM((CH, C), jnp.float32),
                       pltpu.VMEM((CH, C), jnp.float32),
                       pltpu.VMEM((CH, C), jnp.float32),
                       pltpu.VMEM((CH, C), jnp.float32),
                       pltpu.VMEM((CH,), jnp.int32),
                       pltpu.VMEM((CH,), jnp.int32),
                       pltpu.VMEM((CH,), jnp.int32),
                       pltpu.VMEM((CH,), jnp.int32),
                       pltpu.VMEM((16,), jnp.float32),
                       pltpu.SemaphoreType.DMA,
                       pltpu.SemaphoreType.DMA,
                       pltpu.SemaphoreType.DMA,
                       pltpu.SemaphoreType.DMA,
                   ])
def _sc_pass(ptab, r4, rpl, gp_c, sb_c, rb_c, rp_c, out, *rest):
    _sc_body(ptab, r4, rpl, gp_c, sb_c, rb_c, rp_c, out, *rest)


def kernel(p_buffer, ref):
    gp, sb, rb, rp = _CONSTS

    rpl = _tonemap(ref.reshape(B * 3, HW))
    ptabflat, r4flat = _sc_prep(p_buffer.reshape(-1), rpl)
    ptab = ptabflat.reshape(N, C)
    r4 = r4flat.reshape(B * HW, 8)

    out = _sc_pass(ptab, r4, rpl,
                   jnp.asarray(gp), jnp.asarray(sb),
                   jnp.asarray(rb), jnp.asarray(rp))
    return (np.float32(0.125) / np.float32(N)) * jnp.sum(out)


# speedup vs baseline: 31.9918x; 1.6707x over previous
import functools

import jax
import jax.numpy as jnp
import numpy as np
from jax import lax
from jax.experimental import pallas as pl
from jax.experimental.pallas import tpu as pltpu
from jax.experimental.pallas import tpu_sc as plsc

B, S, C, H, W = 8, 8, 8, 128, 128
HW = H * W
SHW = S * HW
N = B * SHW

NC, NS = 2, 16
NW = NC * NS
PER_TILE = N // NW
CH = 1024
CHUNKS = PER_TILE // CH

_GAMMA = np.float32(0.454545)

def _fry_mix(k0, k1, x0, x1):
    rots = ((13, 15, 26, 6), (17, 29, 16, 24))
    ks = (k0, k1, np.uint32(k0 ^ k1 ^ np.uint32(0x1BD11BDA)))
    x0 = x0 + ks[0]
    x1 = x1 + ks[1]
    for i in range(5):
        for r in rots[i % 2]:
            x0 = x0 + x1
            x1 = ((x1 << np.uint32(r)) | (x1 >> np.uint32(32 - r))) ^ x0
        x0 = x0 + ks[(i + 1) % 3]
        x1 = x1 + ks[(i + 2) % 3] + np.uint32(i + 1)
    return x0, x1


def _fry_permutation(seed, n):
    key = (np.uint32(0), np.uint32(seed))
    x = np.arange(n, dtype=np.int32)
    num_rounds = int(np.ceil(3 * np.log(max(1, n)) / np.log(np.iinfo(np.uint32).max)))
    with np.errstate(over='ignore'):
        for _ in range(num_rounds):
            b0, b1 = _fry_mix(key[0], key[1],
                              np.zeros(2, np.uint32), np.arange(2, dtype=np.uint32))
            key, sub = (b0[0], b1[0]), (b0[1], b1[1])
            s0, s1 = _fry_mix(sub[0], sub[1],
                              np.zeros(n, np.uint32), np.arange(n, dtype=np.uint32))
            x = x[np.argsort(s0 ^ s1, kind='stable')]
    return x


def _make_consts():
    sp = _fry_permutation(1, SHW).astype(np.int32)
    sb = _fry_permutation(2, N).astype(np.int32)
    k = np.arange(N, dtype=np.int32)
    gp = (k // SHW) * SHW + sp[k % SHW]
    rp = (sp & (HW - 1)).astype(np.int32)
    rb = ((sb >> 17) << 14) | (sb & (HW - 1))
    return gp, sb.astype(np.int32), rb.astype(np.int32), rp


_CONSTS = _make_consts()


def _tonemap_body(x_ref, o_ref):
    x = jnp.maximum(x_ref[...], 0.0)
    y = x / (1.0 + x)
    t = jnp.exp(_GAMMA * jnp.log(y))
    o_ref[...] = jnp.where(y > 0.0, t, 0.0)


def _tonemap(ref24):
    return pl.pallas_call(
        _tonemap_body,
        out_shape=jax.ShapeDtypeStruct((B * 3, HW), jnp.float32),
    )(ref24)


PC = 2048
PR = 2048


def _sc_prep_body(pflat, rpl, ptab_o, r4_o, ch_v, out_v, r3_v, r8_v, sem):
    cid = lax.axis_index("c")
    sid = lax.axis_index("s")
    wid = sid * NC + cid

    i16 = lax.iota(jnp.int32, 16)
    i8 = i16 * 8

    for b in range(2):
        bs = wid * 2 + b
        base_in = bs * (C * HW)
        base_out = bs * HW * C

        def p_chunk(c, _, base_in=base_in, base_out=base_out):
            p0 = c * PC
            cps = [pltpu.async_copy(
                pflat.at[pl.ds(base_in + ch * HW + p0, PC)],
                ch_v.at[ch], sem) for ch in range(C)]
            for cp in cps:
                cp.wait()

            def step(t, _):
                for ch in range(C):
                    v = ch_v[ch, pl.ds(t * 16, 16)]
                    plsc.store_scatter(out_v, [i8 + (t * 128 + ch)], v)
                return 0

            lax.fori_loop(0, PC // 16, step, 0)
            pltpu.sync_copy(out_v, ptab_o.at[pl.ds(base_out + p0 * 8, PC * 8)])
            return 0

        lax.fori_loop(0, HW // PC, p_chunk, 0)

    bq = wid // 4
    pr0 = (wid % 4) * (HW // 4)

    def r_chunk(c, _):
        p0 = pr0 + c * PR
        cps = [pltpu.async_copy(rpl.at[bq * 3 + ch, pl.ds(p0, PR)],
                                r3_v.at[ch], sem) for ch in range(3)]
        for cp in cps:
            cp.wait()

        def step(t, _):
            for ch in range(3):
                v = r3_v[ch, pl.ds(t * 16, 16)]
                plsc.store_scatter(r8_v, [i8 + (t * 128 + ch)], v)
            return 0

        lax.fori_loop(0, PR // 16, step, 0)
        pltpu.sync_copy(r8_v, r4_o.at[pl.ds((bq * HW + p0) * 8, PR * 8)])
        return 0

    lax.fori_loop(0, (HW // 4) // PR, r_chunk, 0)


@functools.partial(pl.kernel,
                   out_type=(jax.ShapeDtypeStruct((N * C,), jnp.float32),
                             jax.ShapeDtypeStruct((B * HW * 8,), jnp.float32)),
                   mesh=plsc.VectorSubcoreMesh(core_axis_name="c",
                                               subcore_axis_name="s"),
                   compiler_params=pltpu.CompilerParams(
                       needs_layout_passes=False,
                       use_tc_tiling_on_sc=False),
                   scratch_types=[
                       pltpu.VMEM((C, PC), jnp.float32),
                       pltpu.VMEM((PC * 8,), jnp.float32),
                       pltpu.VMEM((3, PR), jnp.float32),
                       pltpu.VMEM((PR * 8,), jnp.float32),
                       pltpu.SemaphoreType.DMA,
                   ])
def _sc_prep(pflat, rpl, ptab_o, r4_o, *rest):
    _sc_prep_body(pflat, rpl, ptab_o, r4_o, *rest)


def _sc_body(ptab, r4, rpl, gp_c, sb_c, rb_c, rp_c, out,
             plane_r, plane_g, plane_b,
             p1_v, p2p_v, p2b_v, rb4_v,
             gp_v, sb_v, rb_v, rp_v, acc_v,
             sem_i, sem_d):
    cid = lax.axis_index("c")
    sid = lax.axis_index("s")
    wid = sid * NC + cid
    k0 = wid * PER_TILE
    bb = wid // (SHW // PER_TILE)
    j0 = (wid % (SHW // PER_TILE)) * PER_TILE

    pltpu.sync_copy(rpl.at[bb * 3 + 0], plane_r)
    pltpu.sync_copy(rpl.at[bb * 3 + 1], plane_g)
    pltpu.sync_copy(rpl.at[bb * 3 + 2], plane_b)

    i16 = lax.iota(jnp.int32, 16)
    c_p = [jnp.full((16,), ch, jnp.int32) for ch in range(C)]
    c_r = [jnp.full((16,), ch, jnp.int32) for ch in range(3)]

    def issue_idx(g, s):
        base_k = k0 + g * CH
        base_j = j0 + g * CH
        pltpu.async_copy(gp_c.at[pl.ds(base_k, CH)], gp_v[s], sem_i[s])
        pltpu.async_copy(sb_c.at[pl.ds(base_k, CH)], sb_v[s], sem_i[s])
        pltpu.async_copy(rb_c.at[pl.ds(base_k, CH)], rb_v[s], sem_i[s])
        pltpu.async_copy(rp_c.at[pl.ds(base_j, CH)], rp_v[s], sem_i[s])

    def wait_idx(s):
        pltpu.make_async_copy(gp_c.at[pl.ds(0, CH)], gp_v[s], sem_i[s]).wait()
        pltpu.make_async_copy(sb_c.at[pl.ds(0, CH)], sb_v[s], sem_i[s]).wait()
        pltpu.make_async_copy(rb_c.at[pl.ds(0, CH)], rb_v[s], sem_i[s]).wait()
        pltpu.make_async_copy(rp_c.at[pl.ds(0, CH)], rp_v[s], sem_i[s]).wait()

    def issue_data(g, s):
        base_k = k0 + g * CH
        pltpu.async_copy(ptab.at[gp_v[s]], p2p_v[s], sem_d[s])
        pltpu.async_copy(ptab.at[sb_v[s]], p2b_v[s], sem_d[s])
        pltpu.async_copy(r4.at[rb_v[s]], rb4_v[s], sem_d[s])
        pltpu.async_copy(ptab.at[pl.ds(base_k, CH)], p1_v[s], sem_d[s])

    def wait_data(s):
        pltpu.make_async_copy(ptab.at[pl.ds(0, CH)], p2p_v[s], sem_d[s]).wait()
        pltpu.make_async_copy(ptab.at[pl.ds(0, CH)], p2b_v[s], sem_d[s]).wait()
        pltpu.make_async_copy(r4.at[pl.ds(0, CH)], rb4_v[s], sem_d[s]).wait()
        pltpu.make_async_copy(ptab.at[pl.ds(0, CH)], p1_v[s], sem_d[s]).wait()

    def compute(g, s, acc):
        base_j = j0 + g * CH
        pixbase = base_j & (HW - 1)

        def step(t, acc):
            r0 = t * 16
            ridx = r0 + i16
            rp16 = rp_v[s][pl.ds(r0, 16)]
            ra_r = plane_r[pl.ds(pixbase + r0, 16)]
            ra_g = plane_g[pl.ds(pixbase + r0, 16)]
            ra_b = plane_b[pl.ds(pixbase + r0, 16)]
            rpr = plsc.load_gather(plane_r, [rp16])
            rpg = plsc.load_gather(plane_g, [rp16])
            rpb = plsc.load_gather(plane_b, [rp16])
            rbr = plsc.load_gather(rb4_v[s], [ridx, c_r[0]])
            rbg = plsc.load_gather(rb4_v[s], [ridx, c_r[1]])
            rbb = plsc.load_gather(rb4_v[s], [ridx, c_r[2]])
            srp = ((ra_r - rpr) * (ra_r - rpr)
                   + (ra_g - rpg) * (ra_g - rpg)
                   + (ra_b - rpb) * (ra_b - rpb))
            srb = ((ra_r - rbr) * (ra_r - rbr)
                   + (ra_g - rbg) * (ra_g - rbg)
                   + (ra_b - rbb) * (ra_b - rbb))
            spp = jnp.zeros((16,), jnp.float32)
            spb = jnp.zeros((16,), jnp.float32)
            for ch in range(C):
                a = plsc.load_gather(p1_v[s], [ridx, c_p[ch]])
                bp = plsc.load_gather(p2p_v[s], [ridx, c_p[ch]])
                bq = plsc.load_gather(p2b_v[s], [ridx, c_p[ch]])
                dp = a - bp
                db = a - bq
                spp = spp + dp * dp
                spb = spb + db * db
            e1 = spp - srp
            e2 = spb - srb
            return acc + (e1 * e1 + e2 * e2)

        return lax.fori_loop(0, CH // 16, step, acc)

    issue_idx(0, 0)
    wait_idx(0)
    issue_data(0, 0)
    issue_idx(1, 1)

    def pair_body(gg, acc):
        for sub in (0, 1):
            g = gg * 2 + sub
            s = sub
            o = 1 - sub

            @pl.when(g + 1 < CHUNKS)
            def _():
                wait_idx(o)
                issue_data(g + 1, o)

            @pl.when(g + 2 < CHUNKS)
            def _():
                issue_idx(g + 2, s)

            wait_data(s)
            acc = compute(g, s, acc)
        return acc

    acc = lax.fori_loop(0, CHUNKS // 2, pair_body,
                        jnp.zeros((16,), jnp.float32))
    acc_v[...] = acc
    pltpu.sync_copy(acc_v, out.at[wid])


_MAIN_SCRATCH = (
    [pltpu.VMEM((HW,), jnp.float32)] * 3
    + [pltpu.VMEM((CH, C), jnp.float32)] * 8
    + [pltpu.VMEM((CH,), jnp.int32)] * 8
    + [pltpu.VMEM((16,), jnp.float32)]
    + [pltpu.SemaphoreType.DMA] * 4
)


@functools.partial(pl.kernel,
                   out_type=jax.ShapeDtypeStruct((NW, 16), jnp.float32),
                   mesh=plsc.VectorSubcoreMesh(core_axis_name="c",
                                               subcore_axis_name="s"),
                   compiler_params=pltpu.CompilerParams(
                       needs_layout_passes=False,
                       use_tc_tiling_on_sc=False),
                   scratch_types=_MAIN_SCRATCH)
def _sc_pass(ptab, r4, rpl, gp_c, sb_c, rb_c, rp_c, out,
             pr, pg, pb,
             p1a, p1b, p2pa, p2pb, p2ba, p2bb, rb4a, rb4b,
             gpa, gpb, sba, sbb, rba, rbb2, rpa, rpb2,
             accv, si0, si1, sd0, sd1):
    _sc_body(ptab, r4, rpl, gp_c, sb_c, rb_c, rp_c, out,
             pr, pg, pb,
             (p1a, p1b), (p2pa, p2pb), (p2ba, p2bb), (rb4a, rb4b),
             (gpa, gpb), (sba, sbb), (rba, rbb2), (rpa, rpb2),
             accv, (si0, si1), (sd0, sd1))


def kernel(p_buffer, ref):
    gp, sb, rb, rp = _CONSTS

    rpl = _tonemap(ref.reshape(B * 3, HW))
    ptabflat, r4flat = _sc_prep(p_buffer.reshape(-1), rpl)
    ptab = ptabflat.reshape(N, C)
    r4 = r4flat.reshape(B * HW, 8)

    out = _sc_pass(ptab, r4, rpl,
                   jnp.asarray(gp), jnp.asarray(sb),
                   jnp.asarray(rb), jnp.asarray(rp))
    return (np.float32(0.125) / np.float32(N)) * jnp.sum(out)


# speedup vs baseline: 35.5828x; 1.1122x over previous
import functools

import jax
import jax.numpy as jnp
import numpy as np
from jax import lax
from jax.experimental import pallas as pl
from jax.experimental.pallas import tpu as pltpu
from jax.experimental.pallas import tpu_sc as plsc

B, S, C, H, W = 8, 8, 8, 128, 128
HW = H * W
SHW = S * HW
N = B * SHW

NC, NS = 2, 16
NW = NC * NS
PER_TILE = N // NW
CH = 1024
CHUNKS = PER_TILE // CH

_GAMMA = np.float32(0.454545)

def _fry_mix(k0, k1, x0, x1):
    rots = ((13, 15, 26, 6), (17, 29, 16, 24))
    ks = (k0, k1, np.uint32(k0 ^ k1 ^ np.uint32(0x1BD11BDA)))
    x0 = x0 + ks[0]
    x1 = x1 + ks[1]
    for i in range(5):
        for r in rots[i % 2]:
            x0 = x0 + x1
            x1 = ((x1 << np.uint32(r)) | (x1 >> np.uint32(32 - r))) ^ x0
        x0 = x0 + ks[(i + 1) % 3]
        x1 = x1 + ks[(i + 2) % 3] + np.uint32(i + 1)
    return x0, x1


def _fry_permutation(seed, n):
    key = (np.uint32(0), np.uint32(seed))
    x = np.arange(n, dtype=np.int32)
    num_rounds = int(np.ceil(3 * np.log(max(1, n)) / np.log(np.iinfo(np.uint32).max)))
    with np.errstate(over='ignore'):
        for _ in range(num_rounds):
            b0, b1 = _fry_mix(key[0], key[1],
                              np.zeros(2, np.uint32), np.arange(2, dtype=np.uint32))
            key, sub = (b0[0], b1[0]), (b0[1], b1[1])
            s0, s1 = _fry_mix(sub[0], sub[1],
                              np.zeros(n, np.uint32), np.arange(n, dtype=np.uint32))
            x = x[np.argsort(s0 ^ s1, kind='stable')]
    return x


def _make_consts():
    sp = _fry_permutation(1, SHW).astype(np.int32)
    sb = _fry_permutation(2, N).astype(np.int32)
    k = np.arange(N, dtype=np.int32)
    gp = (k // SHW) * SHW + sp[k % SHW]
    rp = (sp & (HW - 1)).astype(np.int32)
    rb = ((sb >> 17) << 14) | (sb & (HW - 1))
    return gp, sb.astype(np.int32), rb.astype(np.int32), rp


_CONSTS = _make_consts()


def _tonemap_body(x_ref, o_ref):
    x = jnp.maximum(x_ref[...], 0.0)
    y = x / (1.0 + x)
    t = jnp.exp(_GAMMA * jnp.log(y))
    o_ref[...] = jnp.where(y > 0.0, t, 0.0)


def _tonemap(ref24):
    return pl.pallas_call(
        _tonemap_body,
        out_shape=jax.ShapeDtypeStruct((B * 3, HW), jnp.float32),
    )(ref24)


PC = 2048
PR = 2048


def _sc_prep_body(pflat, rpl, ptab_o, r4_o, ch_v, out_v, r3_v, r8_v,
                  sem_i, sem_o, sem_r):
    cid = lax.axis_index("c")
    sid = lax.axis_index("s")
    wid = sid * NC + cid

    i16 = lax.iota(jnp.int32, 16)
    i8 = i16 * 8

    NCHUNK = 2 * (HW // PC)

    def in_off(c, ch):
        b = c // (HW // PC)
        cc = c % (HW // PC)
        return (wid * 2 + b) * (C * HW) + ch * HW + cc * PC

    def out_off(c):
        b = c // (HW // PC)
        cc = c % (HW // PC)
        return (wid * 2 + b) * HW * C + cc * PC * 8

    def issue_in(c, s):
        for ch in range(C):
            pltpu.async_copy(pflat.at[pl.ds(in_off(c, ch), PC)],
                             ch_v[s].at[ch], sem_i[s])

    def wait_in(s):
        for ch in range(C):
            pltpu.make_async_copy(pflat.at[pl.ds(0, PC)],
                                  ch_v[s].at[ch], sem_i[s]).wait()

    def interleave(s):
        def step(t, _):
            for ch in range(C):
                v = ch_v[s][ch, pl.ds(t * 16, 16)]
                plsc.store_scatter(out_v[s], [i8 + (t * 128 + ch)], v)
            return 0

        lax.fori_loop(0, PC // 16, step, 0)

    def issue_out(c, s):
        pltpu.async_copy(out_v[s], ptab_o.at[pl.ds(out_off(c), PC * 8)],
                         sem_o[s])

    def wait_out(s):
        pltpu.make_async_copy(out_v[s], ptab_o.at[pl.ds(0, PC * 8)],
                              sem_o[s]).wait()

    issue_in(0, 0)

    def pair_body(gg, _):
        for sub in (0, 1):
            c = gg * 2 + sub
            s = sub

            @pl.when(c + 1 < NCHUNK)
            def _():
                issue_in(c + 1, 1 - s)

            wait_in(s)

            @pl.when(c >= 2)
            def _():
                wait_out(s)

            interleave(s)
            issue_out(c, s)
        return 0

    lax.fori_loop(0, NCHUNK // 2, pair_body, 0)
    wait_out(0)
    wait_out(1)

    bq = wid // 4
    pr0 = (wid % 4) * (HW // 4)

    def r_chunk(c, _):
        p0 = pr0 + c * PR
        cps = [pltpu.async_copy(rpl.at[bq * 3 + ch, pl.ds(p0, PR)],
                                r3_v.at[ch], sem_r) for ch in range(3)]
        for cp in cps:
            cp.wait()

        def step(t, _):
            for ch in range(3):
                v = r3_v[ch, pl.ds(t * 16, 16)]
                plsc.store_scatter(r8_v, [i8 + (t * 128 + ch)], v)
            return 0

        lax.fori_loop(0, PR // 16, step, 0)
        pltpu.sync_copy(r8_v, r4_o.at[pl.ds((bq * HW + p0) * 8, PR * 8)])
        return 0

    lax.fori_loop(0, (HW // 4) // PR, r_chunk, 0)


@functools.partial(pl.kernel,
                   out_type=(jax.ShapeDtypeStruct((N * C,), jnp.float32),
                             jax.ShapeDtypeStruct((B * HW * 8,), jnp.float32)),
                   mesh=plsc.VectorSubcoreMesh(core_axis_name="c",
                                               subcore_axis_name="s"),
                   compiler_params=pltpu.CompilerParams(
                       needs_layout_passes=False,
                       use_tc_tiling_on_sc=False),
                   scratch_types=[
                       pltpu.VMEM((C, PC), jnp.float32),
                       pltpu.VMEM((C, PC), jnp.float32),
                       pltpu.VMEM((PC * 8,), jnp.float32),
                       pltpu.VMEM((PC * 8,), jnp.float32),
                       pltpu.VMEM((3, PR), jnp.float32),
                       pltpu.VMEM((PR * 8,), jnp.float32),
                       pltpu.SemaphoreType.DMA,
                       pltpu.SemaphoreType.DMA,
                       pltpu.SemaphoreType.DMA,
                       pltpu.SemaphoreType.DMA,
                       pltpu.SemaphoreType.DMA,
                   ])
def _sc_prep(pflat, rpl, ptab_o, r4_o,
             cva, cvb, ova, ovb, r3v, r8v, sia, sib, soa, sob, sr):
    _sc_prep_body(pflat, rpl, ptab_o, r4_o,
                  (cva, cvb), (ova, ovb), r3v, r8v,
                  (sia, sib), (soa, sob), sr)


def _sc_body(ptab, r4, rpl, gp_c, sb_c, rb_c, rp_c, out,
             plane_r, plane_g, plane_b,
             p1_v, p2p_v, p2b_v, rb4_v,
             gp_v, sb_v, rb_v, rp_v, acc_v,
             sem_i, sem_d):
    cid = lax.axis_index("c")
    sid = lax.axis_index("s")
    wid = sid * NC + cid
    k0 = wid * PER_TILE
    bb = wid // (SHW // PER_TILE)
    j0 = (wid % (SHW // PER_TILE)) * PER_TILE

    pltpu.sync_copy(rpl.at[bb * 3 + 0], plane_r)
    pltpu.sync_copy(rpl.at[bb * 3 + 1], plane_g)
    pltpu.sync_copy(rpl.at[bb * 3 + 2], plane_b)

    i16 = lax.iota(jnp.int32, 16)
    c_p = [jnp.full((16,), ch, jnp.int32) for ch in range(C)]
    c_r = [jnp.full((16,), ch, jnp.int32) for ch in range(3)]

    def issue_idx(g, s):
        base_k = k0 + g * CH
        base_j = j0 + g * CH
        pltpu.async_copy(gp_c.at[pl.ds(base_k, CH)], gp_v[s], sem_i[s])
        pltpu.async_copy(sb_c.at[pl.ds(base_k, CH)], sb_v[s], sem_i[s])
        pltpu.async_copy(rb_c.at[pl.ds(base_k, CH)], rb_v[s], sem_i[s])
        pltpu.async_copy(rp_c.at[pl.ds(base_j, CH)], rp_v[s], sem_i[s])

    def wait_idx(s):
        pltpu.make_async_copy(gp_c.at[pl.ds(0, CH)], gp_v[s], sem_i[s]).wait()
        pltpu.make_async_copy(sb_c.at[pl.ds(0, CH)], sb_v[s], sem_i[s]).wait()
        pltpu.make_async_copy(rb_c.at[pl.ds(0, CH)], rb_v[s], sem_i[s]).wait()
        pltpu.make_async_copy(rp_c.at[pl.ds(0, CH)], rp_v[s], sem_i[s]).wait()

    def issue_data(g, s):
        base_k = k0 + g * CH
        pltpu.async_copy(ptab.at[gp_v[s]], p2p_v[s], sem_d[s])
        pltpu.async_copy(ptab.at[sb_v[s]], p2b_v[s], sem_d[s])
        pltpu.async_copy(r4.at[rb_v[s]], rb4_v[s], sem_d[s])
        pltpu.async_copy(ptab.at[pl.ds(base_k, CH)], p1_v[s], sem_d[s])

    def wait_data(s):
        pltpu.make_async_copy(ptab.at[pl.ds(0, CH)], p2p_v[s], sem_d[s]).wait()
        pltpu.make_async_copy(ptab.at[pl.ds(0, CH)], p2b_v[s], sem_d[s]).wait()
        pltpu.make_async_copy(r4.at[pl.ds(0, CH)], rb4_v[s], sem_d[s]).wait()
        pltpu.make_async_copy(ptab.at[pl.ds(0, CH)], p1_v[s], sem_d[s]).wait()

    def compute(g, s, acc):
        base_j = j0 + g * CH
        pixbase = base_j & (HW - 1)

        def step(t, acc):
            r0 = t * 16
            ridx = r0 + i16
            rp16 = rp_v[s][pl.ds(r0, 16)]
            ra_r = plane_r[pl.ds(pixbase + r0, 16)]
            ra_g = plane_g[pl.ds(pixbase + r0, 16)]
            ra_b = plane_b[pl.ds(pixbase + r0, 16)]
            rpr = plsc.load_gather(plane_r, [rp16])
            rpg = plsc.load_gather(plane_g, [rp16])
            rpb = plsc.load_gather(plane_b, [rp16])
            rbr = plsc.load_gather(rb4_v[s], [ridx, c_r[0]])
            rbg = plsc.load_gather(rb4_v[s], [ridx, c_r[1]])
            rbb = plsc.load_gather(rb4_v[s], [ridx, c_r[2]])
            srp = ((ra_r - rpr) * (ra_r - rpr)
                   + (ra_g - rpg) * (ra_g - rpg)
                   + (ra_b - rpb) * (ra_b - rpb))
            srb = ((ra_r - rbr) * (ra_r - rbr)
                   + (ra_g - rbg) * (ra_g - rbg)
                   + (ra_b - rbb) * (ra_b - rbb))
            spp = jnp.zeros((16,), jnp.float32)
            spb = jnp.zeros((16,), jnp.float32)
            for ch in range(C):
                a = plsc.load_gather(p1_v[s], [ridx, c_p[ch]])
                bp = plsc.load_gather(p2p_v[s], [ridx, c_p[ch]])
                bq = plsc.load_gather(p2b_v[s], [ridx, c_p[ch]])
                dp = a - bp
                db = a - bq
                spp = spp + dp * dp
                spb = spb + db * db
            e1 = spp - srp
            e2 = spb - srb
            return acc + (e1 * e1 + e2 * e2)

        return lax.fori_loop(0, CH // 16, step, acc)

    issue_idx(0, 0)
    wait_idx(0)
    issue_data(0, 0)
    issue_idx(1, 1)

    def pair_body(gg, acc):
        for sub in (0, 1):
            g = gg * 2 + sub
            s = sub
            o = 1 - sub

            @pl.when(g + 1 < CHUNKS)
            def _():
                wait_idx(o)
                issue_data(g + 1, o)

            @pl.when(g + 2 < CHUNKS)
            def _():
                issue_idx(g + 2, s)

            wait_data(s)
            acc = compute(g, s, acc)
        return acc

    acc = lax.fori_loop(0, CHUNKS // 2, pair_body,
                        jnp.zeros((16,), jnp.float32))
    acc_v[...] = acc
    pltpu.sync_copy(acc_v, out.at[wid])


_MAIN_SCRATCH = (
    [pltpu.VMEM((HW,), jnp.float32)] * 3
    + [pltpu.VMEM((CH, C), jnp.float32)] * 8
    + [pltpu.VMEM((CH,), jnp.int32)] * 8
    + [pltpu.VMEM((16,), jnp.float32)]
    + [pltpu.SemaphoreType.DMA] * 4
)


@functools.partial(pl.kernel,
                   out_type=jax.ShapeDtypeStruct((NW, 16), jnp.float32),
                   mesh=plsc.VectorSubcoreMesh(core_axis_name="c",
                                               subcore_axis_name="s"),
                   compiler_params=pltpu.CompilerParams(
                       needs_layout_passes=False,
                       use_tc_tiling_on_sc=False),
                   scratch_types=_MAIN_SCRATCH)
def _sc_pass(ptab, r4, rpl, gp_c, sb_c, rb_c, rp_c, out,
             pr, pg, pb,
             p1a, p1b, p2pa, p2pb, p2ba, p2bb, rb4a, rb4b,
             gpa, gpb, sba, sbb, rba, rbb2, rpa, rpb2,
             accv, si0, si1, sd0, sd1):
    _sc_body(ptab, r4, rpl, gp_c, sb_c, rb_c, rp_c, out,
             pr, pg, pb,
             (p1a, p1b), (p2pa, p2pb), (p2ba, p2bb), (rb4a, rb4b),
             (gpa, gpb), (sba, sbb), (rba, rbb2), (rpa, rpb2),
             accv, (si0, si1), (sd0, sd1))


def kernel(p_buffer, ref):
    gp, sb, rb, rp = _CONSTS

    rpl = _tonemap(ref.reshape(B * 3, HW))
    ptabflat, r4flat = _sc_prep(p_buffer.reshape(-1), rpl)
    ptab = ptabflat.reshape(N, C)
    r4 = r4flat.reshape(B * HW, 8)

    out = _sc_pass(ptab, r4, rpl,
                   jnp.asarray(gp), jnp.asarray(sb),
                   jnp.asarray(rb), jnp.asarray(rp))
    return (np.float32(0.125) / np.float32(N)) * jnp.sum(out)


# speedup vs baseline: 41.1037x; 1.1552x over previous
import functools

import jax
import jax.numpy as jnp
import numpy as np
from jax import lax
from jax.experimental import pallas as pl
from jax.experimental.pallas import tpu as pltpu
from jax.experimental.pallas import tpu_sc as plsc

B, S, C, H, W = 8, 8, 8, 128, 128
HW = H * W
SHW = S * HW
N = B * SHW

NC, NS = 2, 16
NW = NC * NS
PER_TILE = N // NW
CH = 1024
CHUNKS = PER_TILE // CH

_GAMMA = np.float32(0.454545)

def _fry_mix(k0, k1, x0, x1):
    rots = ((13, 15, 26, 6), (17, 29, 16, 24))
    ks = (k0, k1, np.uint32(k0 ^ k1 ^ np.uint32(0x1BD11BDA)))
    x0 = x0 + ks[0]
    x1 = x1 + ks[1]
    for i in range(5):
        for r in rots[i % 2]:
            x0 = x0 + x1
            x1 = ((x1 << np.uint32(r)) | (x1 >> np.uint32(32 - r))) ^ x0
        x0 = x0 + ks[(i + 1) % 3]
        x1 = x1 + ks[(i + 2) % 3] + np.uint32(i + 1)
    return x0, x1


def _fry_permutation(seed, n):
    key = (np.uint32(0), np.uint32(seed))
    x = np.arange(n, dtype=np.int32)
    num_rounds = int(np.ceil(3 * np.log(max(1, n)) / np.log(np.iinfo(np.uint32).max)))
    with np.errstate(over='ignore'):
        for _ in range(num_rounds):
            b0, b1 = _fry_mix(key[0], key[1],
                              np.zeros(2, np.uint32), np.arange(2, dtype=np.uint32))
            key, sub = (b0[0], b1[0]), (b0[1], b1[1])
            s0, s1 = _fry_mix(sub[0], sub[1],
                              np.zeros(n, np.uint32), np.arange(n, dtype=np.uint32))
            x = x[np.argsort(s0 ^ s1, kind='stable')]
    return x


def _make_consts():
    sp = _fry_permutation(1, SHW).astype(np.int32)
    sb = _fry_permutation(2, N).astype(np.int32)
    return sp, sb


_CONSTS = _make_consts()


def _tonemap_body(x_ref, o_ref):
    x = jnp.maximum(x_ref[...], 0.0)
    y = x / (1.0 + x)
    t = jnp.exp(_GAMMA * jnp.log(y))
    o_ref[...] = jnp.where(y > 0.0, t, 0.0)


def _tonemap(ref24):
    return pl.pallas_call(
        _tonemap_body,
        out_shape=jax.ShapeDtypeStruct((B * 3, HW), jnp.float32),
    )(ref24)


PC = 2048
PR = 2048


def _sc_prep_body(pflat, rpl, ptab_o, r4_o, ch_v, out_v, r3_v, r8_v,
                  sem_i, sem_o, sem_r):
    cid = lax.axis_index("c")
    sid = lax.axis_index("s")
    wid = sid * NC + cid

    i16 = lax.iota(jnp.int32, 16)
    i8 = i16 * 8

    NCHUNK = 2 * (HW // PC)

    def in_off(c, ch):
        b = c // (HW // PC)
        cc = c % (HW // PC)
        return (wid * 2 + b) * (C * HW) + ch * HW + cc * PC

    def out_off(c):
        b = c // (HW // PC)
        cc = c % (HW // PC)
        return (wid * 2 + b) * HW * C + cc * PC * 8

    def issue_in(c, s):
        for ch in range(C):
            pltpu.async_copy(pflat.at[pl.ds(in_off(c, ch), PC)],
                             ch_v[s].at[ch], sem_i[s])

    def wait_in(s):
        for ch in range(C):
            pltpu.make_async_copy(pflat.at[pl.ds(0, PC)],
                                  ch_v[s].at[ch], sem_i[s]).wait()

    def interleave(s):
        def step(t, _):
            for ch in range(C):
                v = ch_v[s][ch, pl.ds(t * 16, 16)]
                plsc.store_scatter(out_v[s], [i8 + (t * 128 + ch)], v)
            return 0

        lax.fori_loop(0, PC // 16, step, 0)

    def issue_out(c, s):
        pltpu.async_copy(out_v[s], ptab_o.at[pl.ds(out_off(c), PC * 8)],
                         sem_o[s])

    def wait_out(s):
        pltpu.make_async_copy(out_v[s], ptab_o.at[pl.ds(0, PC * 8)],
                              sem_o[s]).wait()

    issue_in(0, 0)

    def pair_body(gg, _):
        for sub in (0, 1):
            c = gg * 2 + sub
            s = sub

            @pl.when(c + 1 < NCHUNK)
            def _():
                issue_in(c + 1, 1 - s)

            wait_in(s)

            @pl.when(c >= 2)
            def _():
                wait_out(s)

            interleave(s)
            issue_out(c, s)
        return 0

    lax.fori_loop(0, NCHUNK // 2, pair_body, 0)
    wait_out(0)
    wait_out(1)

    bq = wid // 4
    pr0 = (wid % 4) * (HW // 4)

    def r_chunk(c, _):
        p0 = pr0 + c * PR
        cps = [pltpu.async_copy(rpl.at[bq * 3 + ch, pl.ds(p0, PR)],
                                r3_v.at[ch], sem_r) for ch in range(3)]
        for cp in cps:
            cp.wait()

        def step(t, _):
            for ch in range(3):
                v = r3_v[ch, pl.ds(t * 16, 16)]
                plsc.store_scatter(r8_v, [i8 + (t * 128 + ch)], v)
            return 0

        lax.fori_loop(0, PR // 16, step, 0)
        pltpu.sync_copy(r8_v, r4_o.at[pl.ds((bq * HW + p0) * 8, PR * 8)])
        return 0

    lax.fori_loop(0, (HW // 4) // PR, r_chunk, 0)


@functools.partial(pl.kernel,
                   out_type=(jax.ShapeDtypeStruct((N * C,), jnp.float32),
                             jax.ShapeDtypeStruct((B * HW * 8,), jnp.float32)),
                   mesh=plsc.VectorSubcoreMesh(core_axis_name="c",
                                               subcore_axis_name="s"),
                   compiler_params=pltpu.CompilerParams(
                       needs_layout_passes=False,
                       use_tc_tiling_on_sc=False),
                   scratch_types=[
                       pltpu.VMEM((C, PC), jnp.float32),
                       pltpu.VMEM((C, PC), jnp.float32),
                       pltpu.VMEM((PC * 8,), jnp.float32),
                       pltpu.VMEM((PC * 8,), jnp.float32),
                       pltpu.VMEM((3, PR), jnp.float32),
                       pltpu.VMEM((PR * 8,), jnp.float32),
                       pltpu.SemaphoreType.DMA,
                       pltpu.SemaphoreType.DMA,
                       pltpu.SemaphoreType.DMA,
                       pltpu.SemaphoreType.DMA,
                       pltpu.SemaphoreType.DMA,
                   ])
def _sc_prep(pflat, rpl, ptab_o, r4_o,
             cva, cvb, ova, ovb, r3v, r8v, sia, sib, soa, sob, sr):
    _sc_prep_body(pflat, rpl, ptab_o, r4_o,
                  (cva, cvb), (ova, ovb), r3v, r8v,
                  (sia, sib), (soa, sob), sr)


def _sc_body(ptab, r4, rpl, sp_c, sb_c, out,
             plane_r, plane_g, plane_b,
             p1_v, p2p_v, p2b_v, rb4_v,
             sp_v, sb_v, gp_f, sb_f, rb_f, acc_v,
             sem_i, sem_d):
    cid = lax.axis_index("c")
    sid = lax.axis_index("s")
    wid = sid * NC + cid
    k0 = wid * PER_TILE
    bb = wid // (SHW // PER_TILE)
    j0 = (wid % (SHW // PER_TILE)) * PER_TILE
    bb_shw = bb * SHW

    pltpu.sync_copy(rpl.at[bb * 3 + 0], plane_r)
    pltpu.sync_copy(rpl.at[bb * 3 + 1], plane_g)
    pltpu.sync_copy(rpl.at[bb * 3 + 2], plane_b)

    i16 = lax.iota(jnp.int32, 16)
    i16d8 = i16 // 8
    i16m8 = i16 & 7
    c_p = [jnp.full((16,), ch, jnp.int32) for ch in range(C)]
    c_r = [jnp.full((16,), ch, jnp.int32) for ch in range(3)]

    def issue_idx(g, s):
        rk = (k0 + g * CH) // 8
        rj = (j0 + g * CH) // 8
        pltpu.async_copy(sp_c.at[pl.ds(rj, CH // 8)], sp_v[s], sem_i[s])
        pltpu.async_copy(sb_c.at[pl.ds(rk, CH // 8)], sb_v[s], sem_i[s])

    def wait_idx(s):
        pltpu.make_async_copy(sp_c.at[pl.ds(0, CH // 8)], sp_v[s], sem_i[s]).wait()
        pltpu.make_async_copy(sb_c.at[pl.ds(0, CH // 8)], sb_v[s], sem_i[s]).wait()

    def build_idx(s):
        def step(t, _):
            r0 = t * 16
            rhi = r0 // 8 + i16d8
            spv = plsc.load_gather(sp_v[s], [rhi, i16m8])
            sbv = plsc.load_gather(sb_v[s], [rhi, i16m8])
            gp_f[s][pl.ds(r0, 16)] = spv + bb_shw
            sb_f[s][pl.ds(r0, 16)] = sbv
            rb_f[s][pl.ds(r0, 16)] = (((sbv >> 17) << 14) | (sbv & (HW - 1)))
            return 0

        lax.fori_loop(0, CH // 16, step, 0)

    def issue_data(g, s):
        base_k = k0 + g * CH
        pltpu.async_copy(ptab.at[gp_f[s]], p2p_v[s], sem_d[s])
        pltpu.async_copy(ptab.at[sb_f[s]], p2b_v[s], sem_d[s])
        pltpu.async_copy(r4.at[rb_f[s]], rb4_v[s], sem_d[s])
        pltpu.async_copy(ptab.at[pl.ds(base_k, CH)], p1_v[s], sem_d[s])

    def wait_data(s):
        pltpu.make_async_copy(ptab.at[pl.ds(0, CH)], p2p_v[s], sem_d[s]).wait()
        pltpu.make_async_copy(ptab.at[pl.ds(0, CH)], p2b_v[s], sem_d[s]).wait()
        pltpu.make_async_copy(r4.at[pl.ds(0, CH)], rb4_v[s], sem_d[s]).wait()
        pltpu.make_async_copy(ptab.at[pl.ds(0, CH)], p1_v[s], sem_d[s]).wait()

    def compute(g, s, acc):
        base_j = j0 + g * CH
        pixbase = base_j & (HW - 1)

        def step(t, acc):
            r0 = t * 16
            ridx = r0 + i16
            rp16 = plsc.load_gather(sp_v[s], [r0 // 8 + i16d8, i16m8]) & (HW - 1)
            ra_r = plane_r[pl.ds(pixbase + r0, 16)]
            ra_g = plane_g[pl.ds(pixbase + r0, 16)]
            ra_b = plane_b[pl.ds(pixbase + r0, 16)]
            rpr = plsc.load_gather(plane_r, [rp16])
            rpg = plsc.load_gather(plane_g, [rp16])
            rpb = plsc.load_gather(plane_b, [rp16])
            rbr = plsc.load_gather(rb4_v[s], [ridx, c_r[0]])
            rbg = plsc.load_gather(rb4_v[s], [ridx, c_r[1]])
            rbb = plsc.load_gather(rb4_v[s], [ridx, c_r[2]])
            srp = ((ra_r - rpr) * (ra_r - rpr)
                   + (ra_g - rpg) * (ra_g - rpg)
                   + (ra_b - rpb) * (ra_b - rpb))
            srb = ((ra_r - rbr) * (ra_r - rbr)
                   + (ra_g - rbg) * (ra_g - rbg)
                   + (ra_b - rbb) * (ra_b - rbb))
            spp = jnp.zeros((16,), jnp.float32)
            spb = jnp.zeros((16,), jnp.float32)
            for ch in range(C):
                a = plsc.load_gather(p1_v[s], [ridx, c_p[ch]])
                bp = plsc.load_gather(p2p_v[s], [ridx, c_p[ch]])
                bq = plsc.load_gather(p2b_v[s], [ridx, c_p[ch]])
                dp = a - bp
                db = a - bq
                spp = spp + dp * dp
                spb = spb + db * db
            e1 = spp - srp
            e2 = spb - srb
            return acc + (e1 * e1 + e2 * e2)

        return lax.fori_loop(0, CH // 16, step, acc)

    issue_idx(0, 0)
    wait_idx(0)
    build_idx(0)
    issue_data(0, 0)
    issue_idx(1, 1)

    def pair_body(gg, acc):
        for sub in (0, 1):
            g = gg * 2 + sub
            s = sub
            o = 1 - sub

            @pl.when(g + 1 < CHUNKS)
            def _():
                wait_idx(o)
                build_idx(o)
                issue_data(g + 1, o)

            @pl.when(g + 2 < CHUNKS)
            def _():
                issue_idx(g + 2, s)

            wait_data(s)
            acc = compute(g, s, acc)
        return acc

    acc = lax.fori_loop(0, CHUNKS // 2, pair_body,
                        jnp.zeros((16,), jnp.float32))
    acc_v[...] = acc
    pltpu.sync_copy(acc_v, out.at[wid])


_MAIN_SCRATCH = (
    [pltpu.VMEM((HW,), jnp.float32)] * 3
    + [pltpu.VMEM((CH, C), jnp.float32)] * 8
    + [pltpu.VMEM((CH // 8, 8), jnp.int32)] * 4
    + [pltpu.VMEM((CH,), jnp.int32)] * 6
    + [pltpu.VMEM((16,), jnp.float32)]
    + [pltpu.SemaphoreType.DMA] * 4
)


@functools.partial(pl.kernel,
                   out_type=jax.ShapeDtypeStruct((NW, 16), jnp.float32),
                   mesh=plsc.VectorSubcoreMesh(core_axis_name="c",
                                               subcore_axis_name="s"),
                   compiler_params=pltpu.CompilerParams(
                       needs_layout_passes=False,
                       use_tc_tiling_on_sc=False),
                   scratch_types=_MAIN_SCRATCH)
def _sc_pass(ptab, r4, rpl, sp_c, sb_c, out,
             pr, pg, pb,
             p1a, p1b, p2pa, p2pb, p2ba, p2bb, rb4a, rb4b,
             spa, spb2, sba, sbb,
             gpfa, gpfb, sbfa, sbfb, rbfa, rbfb,
             accv, si0, si1, sd0, sd1):
    _sc_body(ptab, r4, rpl, sp_c, sb_c, out,
             pr, pg, pb,
             (p1a, p1b), (p2pa, p2pb), (p2ba, p2bb), (rb4a, rb4b),
             (spa, spb2), (sba, sbb),
             (gpfa, gpfb), (sbfa, sbfb), (rbfa, rbfb),
             accv, (si0, si1), (sd0, sd1))


def kernel(p_buffer, ref):
    sp, sb = _CONSTS

    rpl = _tonemap(ref.reshape(B * 3, HW))
    ptabflat, r4flat = _sc_prep(p_buffer.reshape(-1), rpl)
    ptab = ptabflat.reshape(N, C)
    r4 = r4flat.reshape(B * HW, 8)

    out = _sc_pass(ptab, r4, rpl,
                   jnp.asarray(sp.reshape(-1, 8)), jnp.asarray(sb.reshape(-1, 8)))
    return (np.float32(0.125) / np.float32(N)) * jnp.sum(out)
